# baseline (speedup 1.0000x reference)
import functools

import jax
import jax.numpy as jnp
import numpy as np
from jax import lax
from jax.experimental import pallas as pl
from jax.experimental.pallas import tpu as pltpu
from jax.experimental.pallas import tpu_sc as plsc

BATCH = 4
SEQ = 2048
D = 1024
NC, NS = 2, 16
NW = NC * NS
TOK = BATCH * SEQ
ROWS_PW = TOK // NW
GC = 16
NGC = ROWS_PW // GC
NSLOT = 4
OUT_LAG = 2

BLK = 512
EPS = 1e-5


def _gather_body(idx_hbm, tok_hbm, out_hbm, idx_v, buf, sg0, sg1, sg2, sg3,
                 so0, so1, so2, so3):
    wid = lax.axis_index("s") * NC + lax.axis_index("c")
    base = wid * ROWS_PW
    sg = (sg0, sg1, sg2, sg3)
    so = (so0, so1, so2, so3)

    pltpu.sync_copy(idx_hbm.at[pl.ds(base, ROWS_PW)], idx_v)

    def start_g(c, b):
        pltpu.async_copy(tok_hbm.at[idx_v.at[pl.ds(c * GC, GC)]], buf.at[b],
                         sg[b])

    def wait_g(b):
        pltpu.make_async_copy(tok_hbm.at[pl.ds(0, GC)], buf.at[b],
                              sg[b]).wait()

    def start_o(c, b):
        pltpu.async_copy(buf.at[b], out_hbm.at[pl.ds(base + c * GC, GC)],
                         so[b])

    def wait_o(b):
        pltpu.make_async_copy(buf.at[b], out_hbm.at[pl.ds(0, GC)],
                              so[b]).wait()

    for g in range(NGC + OUT_LAG):
        if g < NGC:
            b = g % NSLOT
            if g >= NSLOT:
                wait_o(b)
            start_g(g, b)
        if g >= OUT_LAG:
            c = g - OUT_LAG
            b2 = c % NSLOT
            wait_g(b2)
            start_o(c, b2)
    for c in range(NGC - NSLOT, NGC):
        wait_o(c % NSLOT)


def _sc_gather(idx, token_table):
    mesh = plsc.VectorSubcoreMesh(core_axis_name="c", subcore_axis_name="s")
    fn = pl.kernel(
        _gather_body,
        out_type=jax.ShapeDtypeStruct((TOK, D), jnp.float32),
        mesh=mesh,
        compiler_params=pltpu.CompilerParams(needs_layout_passes=False),
        scratch_types=[
            pltpu.VMEM((ROWS_PW,), jnp.int32),
            pltpu.VMEM((NSLOT, GC, D), jnp.float32),
            pltpu.SemaphoreType.DMA,
            pltpu.SemaphoreType.DMA,
            pltpu.SemaphoreType.DMA,
            pltpu.SemaphoreType.DMA,
            pltpu.SemaphoreType.DMA,
            pltpu.SemaphoreType.DMA,
            pltpu.SemaphoreType.DMA,
            pltpu.SemaphoreType.DMA,
        ],
    )
    return fn(idx, token_table)


def _ln_block(g_ref, p_ref, gam_ref, bet_ref, o_ref):
    x = g_ref[...] + p_ref[...]
    mean = jnp.mean(x, axis=-1, keepdims=True)
    xc = x - mean
    var = jnp.mean(xc * xc, axis=-1, keepdims=True)
    rstd = lax.rsqrt(var + EPS)
    o_ref[...] = xc * rstd * gam_ref[...] + bet_ref[...]


def _tc_ln(gathered, pos_table, ln_gamma, ln_beta):
    pos_rep = SEQ // BLK
    nbat = TOK // SEQ
    return pl.pallas_call(
        _ln_block,
        grid=(pos_rep, nbat),
        in_specs=[
            pl.BlockSpec((BLK, D), lambda i, j: (j * (SEQ // BLK) + i, 0)),
            pl.BlockSpec((BLK, D), lambda i, j: (i, 0)),
            pl.BlockSpec((1, D), lambda i, j: (0, 0)),
            pl.BlockSpec((1, D), lambda i, j: (0, 0)),
        ],
        out_specs=pl.BlockSpec((BLK, D), lambda i, j: (j * (SEQ // BLK) + i, 0)),
        out_shape=jax.ShapeDtypeStruct((TOK, D), jnp.float32),
    )(gathered, pos_table, ln_gamma.reshape(1, D), ln_beta.reshape(1, D))


@jax.jit
def _run(idx, token_table, pos_table, ln_gamma, ln_beta):
    gathered = _sc_gather(idx, token_table)
    return _tc_ln(gathered, pos_table, ln_gamma, ln_beta)


def kernel(input_token, token_table, pos_table, ln_gamma, ln_beta):
    idx = input_token.reshape(-1).astype(jnp.int32)
    out = _run(idx, token_table, pos_table, ln_gamma, ln_beta)
    return out.reshape(BATCH, SEQ, D)


# speedup vs baseline: 1.0474x; 1.0474x over previous
import functools

import jax
import jax.numpy as jnp
import numpy as np
from jax import lax
from jax.experimental import pallas as pl
from jax.experimental.pallas import tpu as pltpu
from jax.experimental.pallas import tpu_sc as plsc

BATCH = 4
SEQ = 2048
D = 1024
NC, NS = 2, 16
NW = NC * NS
TOK = BATCH * SEQ
ROWS_PW = TOK // NW
GC = 16
NGC = ROWS_PW // GC
NSLOT = 4
OUT_LAG = 2

BLK = 1024
EPS = 1e-5


def _gather_body(idx_hbm, tok_hbm, out_hbm, idx_v, buf, sg0, sg1, sg2, sg3,
                 so0, so1, so2, so3):
    wid = lax.axis_index("s") * NC + lax.axis_index("c")
    base = wid * ROWS_PW
    sg = (sg0, sg1, sg2, sg3)
    so = (so0, so1, so2, so3)

    pltpu.sync_copy(idx_hbm.at[pl.ds(wid * NGC, NGC)], idx_v)

    def start_g(c, b):
        pltpu.async_copy(tok_hbm.at[idx_v.at[c]], buf.at[b], sg[b])

    def wait_g(b):
        pltpu.make_async_copy(tok_hbm.at[pl.ds(0, GC)], buf.at[b],
                              sg[b]).wait()

    def start_o(c, b):
        pltpu.async_copy(buf.at[b], out_hbm.at[pl.ds(base + c * GC, GC)],
                         so[b])

    def wait_o(b):
        pltpu.make_async_copy(buf.at[b], out_hbm.at[pl.ds(0, GC)],
                              so[b]).wait()

    for g in range(NGC + OUT_LAG):
        if g < NGC:
            b = g % NSLOT
            if g >= NSLOT:
                wait_o(b)
            start_g(g, b)
        if g >= OUT_LAG:
            c = g - OUT_LAG
            b2 = c % NSLOT
            wait_g(b2)
            start_o(c, b2)
    for c in range(NGC - NSLOT, NGC):
        wait_o(c % NSLOT)


def _sc_gather(idx, token_table):
    mesh = plsc.VectorSubcoreMesh(core_axis_name="c", subcore_axis_name="s")
    fn = pl.kernel(
        _gather_body,
        out_type=jax.ShapeDtypeStruct((TOK, D), jnp.float32),
        mesh=mesh,
        compiler_params=pltpu.CompilerParams(needs_layout_passes=False),
        scratch_types=[
            pltpu.VMEM((NGC, GC), jnp.int32),
            pltpu.VMEM((NSLOT, GC, D), jnp.float32),
            pltpu.SemaphoreType.DMA,
            pltpu.SemaphoreType.DMA,
            pltpu.SemaphoreType.DMA,
            pltpu.SemaphoreType.DMA,
            pltpu.SemaphoreType.DMA,
            pltpu.SemaphoreType.DMA,
            pltpu.SemaphoreType.DMA,
            pltpu.SemaphoreType.DMA,
        ],
    )
    return fn(idx.reshape(TOK // GC, GC), token_table)


def _ln_block(g_ref, p_ref, gam_ref, bet_ref, o_ref):
    x = g_ref[...] + p_ref[...]
    mean = jnp.mean(x, axis=-1, keepdims=True)
    xc = x - mean
    var = jnp.mean(xc * xc, axis=-1, keepdims=True)
    rstd = lax.rsqrt(var + EPS)
    o_ref[...] = xc * rstd * gam_ref[...] + bet_ref[...]


def _tc_ln(gathered, pos_table, ln_gamma, ln_beta):
    pos_rep = SEQ // BLK
    nbat = TOK // SEQ
    return pl.pallas_call(
        _ln_block,
        grid=(pos_rep, nbat),
        in_specs=[
            pl.BlockSpec((BLK, D), lambda i, j: (j * (SEQ // BLK) + i, 0)),
            pl.BlockSpec((BLK, D), lambda i, j: (i, 0)),
            pl.BlockSpec((1, D), lambda i, j: (0, 0)),
            pl.BlockSpec((1, D), lambda i, j: (0, 0)),
        ],
        out_specs=pl.BlockSpec((BLK, D), lambda i, j: (j * (SEQ // BLK) + i, 0)),
        out_shape=jax.ShapeDtypeStruct((TOK, D), jnp.float32),
    )(gathered, pos_table, ln_gamma.reshape(1, D), ln_beta.reshape(1, D))


@jax.jit
def _run(idx, token_table, pos_table, ln_gamma, ln_beta):
    gathered = _sc_gather(idx, token_table)
    return _tc_ln(gathered, pos_table, ln_gamma, ln_beta)


def kernel(input_token, token_table, pos_table, ln_gamma, ln_beta):
    idx = input_token.reshape(-1).astype(jnp.int32)
    out = _run(idx, token_table, pos_table, ln_gamma, ln_beta)
    return out.reshape(BATCH, SEQ, D)
